# baseline (speedup 1.0000x reference)
import functools

import jax
import jax.numpy as jnp
from jax import lax
from jax.experimental import pallas as pl
from jax.experimental.pallas import tpu as pltpu
from jax.experimental.pallas import tpu_sc as plsc

_N = 1024
_TD = 64
_NB = 22
_MAXD = 40.0
_ROWS = 32

_NC = 2
_NS = 16
_NW = _NC * _NS
_RPW = _N // _NW


def _bins_body(a_ref, cT_ref, e_ref, out_ref):
    a = a_ref[...]
    cT = cT_ref[...]
    e = e_ref[...]

    dx = a[:, 0:1] - cT[0:1, :]
    dy = a[:, 1:2] - cT[1:2, :]
    dz = a[:, 2:3] - cT[2:3, :]
    dist = jnp.sqrt(dx * dx + dy * dy + dz * dz + 1e-8)

    acc = jnp.zeros_like(dist, dtype=jnp.int32)
    for k in range(_NB - 1):
        acc = acc + (dist > e[k, 0]).astype(jnp.int32)
    out_ref[...] = acc


def _sc_expand_body(bins_hbm, table_hbm, out_hbm, idx_v, rows_v, sem):
    wid = lax.axis_index("s") * _NC + lax.axis_index("c")
    base = wid * _RPW

    def row_step(r, carry):
        i = base + r
        pltpu.sync_copy(bins_hbm.at[i], idx_v)
        cps = [
            pltpu.async_copy(
                table_hbm.at[idx_v.at[c]],
                rows_v.at[pl.ds(c * 128, 128)],
                sem,
            )
            for c in range(8)
        ]
        for cp in cps:
            cp.wait()
        pltpu.sync_copy(rows_v, out_hbm.at[i])
        return carry

    lax.fori_loop(0, _RPW, row_step, 0)


_sc_expand = functools.partial(
    pl.kernel,
    mesh=plsc.VectorSubcoreMesh(core_axis_name="c", subcore_axis_name="s"),
    out_type=jax.ShapeDtypeStruct((_N, _N, _TD), jnp.float32),
    compiler_params=pltpu.CompilerParams(use_tc_tiling_on_sc=False),
    scratch_types=[
        pltpu.VMEM((8, 128), jnp.int32),
        pltpu.VMEM((_N, _TD), jnp.float32),
        pltpu.SemaphoreType.DMA,
    ],
)(_sc_expand_body)


def kernel(coords, W, b):
    bin_width = _MAXD / (_NB - 1)
    edges = jnp.arange(0.0, _MAXD + bin_width, bin_width, dtype=jnp.float32)[:_NB]
    ecol = edges[: _NB - 1].reshape(_NB - 1, 1)
    cT = coords.T

    bins = pl.pallas_call(
        _bins_body,
        grid=(_N // _ROWS,),
        in_specs=[
            pl.BlockSpec((_ROWS, 3), lambda i: (i, 0)),
            pl.BlockSpec((3, _N), lambda i: (0, 0)),
            pl.BlockSpec((_NB - 1, 1), lambda i: (0, 0)),
        ],
        out_specs=pl.BlockSpec((_ROWS, _N), lambda i: (i, 0)),
        out_shape=jax.ShapeDtypeStruct((_N, _N), jnp.int32),
    )(coords, cT, ecol)

    bins3 = bins.reshape(_N, 8, 128)
    table = W.T + b[None, :]
    return _sc_expand(bins3, table)


# speedup vs baseline: 151.7341x; 151.7341x over previous
import jax
import jax.numpy as jnp
from jax.experimental import pallas as pl

_N = 1024
_TD = 64
_NB = 22
_MAXD = 40.0
_ROWS = 32


def _body(a_ref, cT_ref, ecol_ref, dT_ref, t0_ref, out_ref):
    a = a_ref[...]
    cT = cT_ref[...]
    ecol = ecol_ref[...]
    dT = dT_ref[...]
    t0 = t0_ref[...]

    dx = a[:, 0:1] - cT[0:1, :]
    dy = a[:, 1:2] - cT[1:2, :]
    dz = a[:, 2:3] - cT[2:3, :]
    d2 = dx * dx + dy * dy + dz * dz
    dist = jnp.sqrt(d2 + 1e-8)

    stair = (dist.reshape(1, _ROWS * _N) > ecol).astype(jnp.float32)
    res = jax.lax.dot_general(
        dT, stair,
        dimension_numbers=(((1,), (0,)), ((), ())),
        preferred_element_type=jnp.float32,
    ) + t0
    for r in range(_ROWS):
        out_ref[r] = res[:, r * _N : (r + 1) * _N]


def kernel(coords, W, b):
    bin_width = _MAXD / (_NB - 1)
    edges = jnp.arange(0.0, _MAXD + bin_width, bin_width, dtype=jnp.float32)[:_NB]
    ecol = edges[: _NB - 1].reshape(_NB - 1, 1)

    table = W.T + b[None, :]
    dT = (table[1:, :] - table[:-1, :]).T
    t0 = table[0:1, :].T
    cT = coords.T

    out = pl.pallas_call(
        _body,
        grid=(_N // _ROWS,),
        in_specs=[
            pl.BlockSpec((_ROWS, 3), lambda i: (i, 0)),
            pl.BlockSpec((3, _N), lambda i: (0, 0)),
            pl.BlockSpec((_NB - 1, 1), lambda i: (0, 0)),
            pl.BlockSpec((_TD, _NB - 1), lambda i: (0, 0)),
            pl.BlockSpec((_TD, 1), lambda i: (0, 0)),
        ],
        out_specs=pl.BlockSpec((_ROWS, _TD, _N), lambda i: (i, 0, 0)),
        out_shape=jax.ShapeDtypeStruct((_N, _TD, _N), jnp.float32),
    )(coords, cT, ecol, dT, t0)
    return out.transpose(0, 2, 1)
